# baseline (speedup 1.0000x reference)
import functools

import jax
import jax.numpy as jnp
from jax import lax
from jax.experimental import pallas as pl
from jax.experimental.pallas import tpu as pltpu
from jax.experimental.pallas import tpu_sc as plsc

E, D, C, B = 8, 2048, 1000, 1024
CP = 1024
BLK = 256
NB = B // BLK
T = NB + E - 1

_info = plsc.get_sparse_core_info()
_NC, _NS = _info.num_cores, _info.num_subcores
NW = _NC * _NS
BPW = B // NW

_sc_mesh = plsc.VectorSubcoreMesh(core_axis_name="c", subcore_axis_name="s")


@functools.partial(
    pl.kernel, mesh=_sc_mesh,
    out_type=jax.ShapeDtypeStruct((B, D), jnp.float32),
    scratch_types=[
        pltpu.VMEM((BPW,), jnp.int32),
        pltpu.VMEM((BPW, D), jnp.float32),
        pltpu.SemaphoreType.DMA,
    ],
)
def _sc_dispatch_rows(x_hbm, slot_hbm, out_hbm, idx_v, rows_v, sem):
    wid = lax.axis_index("s") * _NC + lax.axis_index("c")
    base = wid * BPW
    pltpu.sync_copy(slot_hbm.at[pl.ds(base, BPW)], idx_v)
    pltpu.sync_copy(x_hbm.at[pl.ds(base, BPW)], rows_v)
    pltpu.async_copy(rows_v, out_hbm.at[idx_v], sem).wait()


def _mm_body(rb_ref, ex_ref, vl_ref, ic_ref, xs_ref, w_ref, b_ref, slot_ref,
             o_ref, wb_ref):
    t = pl.program_id(0)
    ex = ex_ref[t]
    prev = jnp.maximum(t - 1, 0)
    new_w = jnp.logical_or(t == 0, ex_ref[t] != ex_ref[prev])

    @pl.when(t == 0)
    def _():
        o_ref[...] = jnp.zeros_like(o_ref)

    @pl.when(jnp.logical_and(new_w, vl_ref[t] == 1))
    def _():
        wb_ref[...] = w_ref[0].astype(jnp.bfloat16)

    @pl.when(vl_ref[t] == 1)
    def _():
        base = rb_ref[t] * BLK
        row = base + lax.broadcasted_iota(jnp.int32, (BLK, 1), 0)
        start = jnp.where(ex == 0, 0, ic_ref[jnp.maximum(ex - 1, 0)])
        mask = jnp.logical_and(row >= start, row < ic_ref[ex])
        xm = jnp.where(mask, xs_ref[...], 0.0).astype(jnp.bfloat16)
        y = lax.dot_general(xm, wb_ref[...], (((1,), (1,)), ((), ())),
                            preferred_element_type=jnp.float32)
        y = y + jnp.where(mask, b_ref[0], 0.0)
        pt = (slot_ref[...] == base +
              lax.broadcasted_iota(jnp.int32, (B, BLK), 1)).astype(jnp.bfloat16)
        o_ref[...] += jnp.dot(pt, y.astype(jnp.bfloat16),
                              preferred_element_type=jnp.float32)


def _grouped_mm(xs, W, b, slot2, rbs, exs, vld, incl):
    grid_spec = pltpu.PrefetchScalarGridSpec(
        num_scalar_prefetch=4,
        grid=(T,),
        in_specs=[
            pl.BlockSpec((BLK, D), lambda t, rb, ex, vl, ic: (rb[t], 0)),
            pl.BlockSpec((1, C, D), lambda t, rb, ex, vl, ic: (ex[t], 0, 0)),
            pl.BlockSpec((1, 1, C), lambda t, rb, ex, vl, ic: (ex[t], 0, 0)),
            pl.BlockSpec((B, 1), lambda t, rb, ex, vl, ic: (0, 0)),
        ],
        out_specs=pl.BlockSpec((B, C), lambda t, rb, ex, vl, ic: (0, 0)),
        scratch_shapes=[pltpu.VMEM((C, D), jnp.bfloat16)],
    )
    return pl.pallas_call(
        _mm_body, grid_spec=grid_spec,
        out_shape=jax.ShapeDtypeStruct((B, C), jnp.float32),
    )(rbs, exs, vld, incl, xs, jnp.transpose(W, (0, 2, 1)), b.reshape(E, 1, C),
      slot2)


def _routing(alpha, n):
    d = jax.random.categorical(
        jax.random.key(42), jnp.log(alpha), shape=(n,)).astype(jnp.int32)
    oh = (d[:, None] == jnp.arange(E, dtype=jnp.int32)[None, :]).astype(jnp.int32)
    counts = oh.sum(0)
    incl = jnp.cumsum(counts)
    offs = incl - counts
    pos = jnp.cumsum(oh, axis=0) - oh
    slotmat = (oh * (offs[None, :] + pos)).astype(jnp.int32)
    slot = slotmat.sum(1)
    slot2 = slotmat.sum(1, keepdims=True)
    ds = (jnp.arange(n, dtype=jnp.int32)[:, None] >= incl[None, :]).sum(1).astype(jnp.int32)

    lo = ds[::BLK]
    hi = ds[BLK - 1::BLK]
    npairs = hi - lo + 1
    starts = jnp.concatenate(
        [jnp.zeros((1,), jnp.int32), jnp.cumsum(npairs)[:-1].astype(jnp.int32)])
    total = starts[-1] + npairs[-1]
    t_idx = jnp.arange(T, dtype=jnp.int32)
    rb_t = jnp.clip(jnp.searchsorted(starts, t_idx, side="right").astype(jnp.int32) - 1,
                    0, NB - 1)
    ex_t = jnp.clip(lo[rb_t] + (t_idx - starts[rb_t]), 0, E - 1).astype(jnp.int32)
    vl_t = (t_idx < total).astype(jnp.int32)
    ex_t = jnp.where(vl_t == 1, ex_t, hi[-1])
    return slot, slot2, rb_t, ex_t, vl_t, incl


def kernel(x, W, b, alpha):
    n = x.shape[0]
    slot, slot2, rbs, exs, vld, incl = _routing(alpha, n)
    xs = _sc_dispatch_rows(x, slot)
    return _grouped_mm(xs, W, b, slot2, rbs, exs, vld, incl)


# speedup vs baseline: 1.0129x; 1.0129x over previous
import functools

import jax
import jax.numpy as jnp
from jax import lax
from jax.experimental import pallas as pl
from jax.experimental.pallas import tpu as pltpu
from jax.experimental.pallas import tpu_sc as plsc

E, D, C, B = 8, 2048, 1000, 1024
CP = 1024
BLK = 256
NB = B // BLK
T = NB + E - 1

_info = plsc.get_sparse_core_info()
_NC, _NS = _info.num_cores, _info.num_subcores
NW = _NC * _NS
BPW = B // NW

_sc_mesh = plsc.VectorSubcoreMesh(core_axis_name="c", subcore_axis_name="s")


@functools.partial(
    pl.kernel, mesh=_sc_mesh,
    out_type=jax.ShapeDtypeStruct((B, D), jnp.float32),
    scratch_types=[
        pltpu.VMEM((BPW,), jnp.int32),
        pltpu.VMEM((BPW, D), jnp.float32),
        pltpu.SemaphoreType.DMA,
    ],
)
def _sc_dispatch_rows(x_hbm, slot_hbm, out_hbm, idx_v, rows_v, sem):
    wid = lax.axis_index("s") * _NC + lax.axis_index("c")
    base = wid * BPW
    pltpu.sync_copy(slot_hbm.at[pl.ds(base, BPW)], idx_v)
    pltpu.sync_copy(x_hbm.at[pl.ds(base, BPW)], rows_v)
    pltpu.async_copy(rows_v, out_hbm.at[idx_v], sem).wait()


def _mm_body(rb_ref, ex_ref, vl_ref, ic_ref, xs_ref, w_ref, b_ref, slot_ref,
             o_ref, wb_ref):
    t = pl.program_id(0)
    ex = ex_ref[t]
    prev = jnp.maximum(t - 1, 0)
    new_w = jnp.logical_or(t == 0, ex_ref[t] != ex_ref[prev])

    @pl.when(t == 0)
    def _():
        o_ref[...] = jnp.zeros_like(o_ref)

    @pl.when(jnp.logical_and(new_w, vl_ref[t] == 1))
    def _():
        wb_ref[...] = w_ref[0].astype(jnp.bfloat16)

    @pl.when(vl_ref[t] == 1)
    def _():
        base = rb_ref[t] * BLK
        row = base + lax.broadcasted_iota(jnp.int32, (BLK, 1), 0)
        start = jnp.where(ex == 0, 0, ic_ref[jnp.maximum(ex - 1, 0)])
        mask = jnp.logical_and(row >= start, row < ic_ref[ex])
        xm = jnp.where(mask, xs_ref[...], 0.0).astype(jnp.bfloat16)
        y = lax.dot_general(xm, wb_ref[...], (((1,), (1,)), ((), ())),
                            preferred_element_type=jnp.float32)
        y = y + jnp.where(mask, b_ref[0], 0.0)
        pt = (slot_ref[...] == base +
              lax.broadcasted_iota(jnp.int32, (B, BLK), 1)).astype(jnp.bfloat16)
        o_ref[...] += jnp.dot(pt, y.astype(jnp.bfloat16),
                              preferred_element_type=jnp.float32)


def _grouped_mm(xs, W, b, slot2, rbs, exs, vld, incl):
    grid_spec = pltpu.PrefetchScalarGridSpec(
        num_scalar_prefetch=4,
        grid=(T,),
        in_specs=[
            pl.BlockSpec((BLK, D), lambda t, rb, ex, vl, ic: (rb[t], 0)),
            pl.BlockSpec((1, C, D), lambda t, rb, ex, vl, ic: (ex[t], 0, 0)),
            pl.BlockSpec((1, 1, C), lambda t, rb, ex, vl, ic: (ex[t], 0, 0)),
            pl.BlockSpec((B, 1), lambda t, rb, ex, vl, ic: (0, 0)),
        ],
        out_specs=pl.BlockSpec((B, C), lambda t, rb, ex, vl, ic: (0, 0)),
        scratch_shapes=[pltpu.VMEM((C, D), jnp.bfloat16)],
    )
    return pl.pallas_call(
        _mm_body, grid_spec=grid_spec,
        out_shape=jax.ShapeDtypeStruct((B, C), jnp.float32),
    )(rbs, exs, vld, incl, xs, jnp.transpose(W, (0, 2, 1)), b.reshape(E, 1, C),
      slot2)


def _routing(alpha, n):
    d = jax.random.categorical(
        jax.random.key(42), jnp.log(alpha), shape=(n,)).astype(jnp.int32)
    oh = (d[:, None] == jnp.arange(E, dtype=jnp.int32)[None, :]).astype(jnp.int32)
    counts = oh.sum(0)
    incl = jnp.cumsum(counts)
    offs = incl - counts
    pos = jnp.cumsum(oh, axis=0) - oh
    slotmat = (oh * (offs[None, :] + pos)).astype(jnp.int32)
    slot = slotmat.sum(1)
    slot2 = slotmat.sum(1, keepdims=True)
    ds = (jnp.arange(n, dtype=jnp.int32)[:, None] >= incl[None, :]).sum(1).astype(jnp.int32)

    lo = ds[::BLK]
    hi = ds[BLK - 1::BLK]
    npairs = hi - lo + 1
    starts = jnp.concatenate(
        [jnp.zeros((1,), jnp.int32), jnp.cumsum(npairs)[:-1].astype(jnp.int32)])
    total = starts[-1] + npairs[-1]
    t_idx = jnp.arange(T, dtype=jnp.int32)
    rb_t = jnp.clip((t_idx[:, None] >= starts[None, :]).sum(1).astype(jnp.int32) - 1,
                    0, NB - 1)
    ex_t = jnp.clip(lo[rb_t] + (t_idx - starts[rb_t]), 0, E - 1).astype(jnp.int32)
    vl_t = (t_idx < total).astype(jnp.int32)
    ex_t = jnp.where(vl_t == 1, ex_t, hi[-1])
    return slot, slot2, rb_t, ex_t, vl_t, incl


def kernel(x, W, b, alpha):
    n = x.shape[0]
    slot, slot2, rbs, exs, vld, incl = _routing(alpha, n)
    xs = _sc_dispatch_rows(x, slot)
    return _grouped_mm(xs, W, b, slot2, rbs, exs, vld, incl)
